# baseline (speedup 1.0000x reference)
import functools

import jax
import jax.numpy as jnp
from jax import lax
from jax.experimental import pallas as pl
from jax.experimental.pallas import tpu as pltpu
from jax.experimental.pallas import tpu_sc as plsc

NUM_PAGES = 1024
TOKENS_PER_PAGE = 32
SLOTS = 16
MAX_PAGES_PER_SLOT = 64
NUM_KV_HEADS = 8
HEAD_DIM = 64
NROWS = NUM_KV_HEADS * TOKENS_PER_PAGE * HEAD_DIM
PM_FLAT = SLOTS * MAX_PAGES_PER_SLOT
RESERVE = 32
BLK = 256


def _tc_body(rel_ref, k_ref, v_ref, ko_ref, vo_ref):
    lanes = lax.broadcasted_iota(jnp.int32, (MAX_PAGES_PER_SLOT, NUM_PAGES), 1)
    hit = jnp.max(jnp.where(lanes == rel_ref[...], 1, 0), axis=0, keepdims=True)
    released = hit > 0
    zero = jnp.zeros((), jnp.float32)
    ko_ref[...] = jnp.where(released, zero, k_ref[...])
    vo_ref[...] = jnp.where(released, zero, v_ref[...])


_kv2d = jax.ShapeDtypeStruct((NROWS, NUM_PAGES), jnp.float32)
_tc_call = pl.pallas_call(
    _tc_body,
    out_shape=(_kv2d, _kv2d),
    grid=(NROWS // BLK,),
    in_specs=[
        pl.BlockSpec((MAX_PAGES_PER_SLOT, 1), lambda i: (0, 0)),
        pl.BlockSpec((BLK, NUM_PAGES), lambda i: (i, 0)),
        pl.BlockSpec((BLK, NUM_PAGES), lambda i: (i, 0)),
    ],
    out_specs=[
        pl.BlockSpec((BLK, NUM_PAGES), lambda i: (i, 0)),
        pl.BlockSpec((BLK, NUM_PAGES), lambda i: (i, 0)),
    ],
)


def _gather16(x, idx):
    return x.at[idx].get(mode="promise_in_bounds")


def _prefix16(x):
    lane = lax.iota(jnp.int32, 16)
    zero = jnp.zeros((16,), jnp.int32)
    for k in (1, 2, 4, 8):
        shifted = _gather16(x, jnp.maximum(lane - k, 0))
        x = x + jnp.where(lane >= k, shifted, zero)
    return x


def _splat_last(x):
    return _gather16(x, jnp.full((16,), 15, jnp.int32))


def _sc_body(ps_hbm, rel_hbm, pm_hbm, meta_hbm,
             ps_out, pm_out, meta_out,
             rel_v, pm_v, ps_v, row_v, meta_v):
    cid = lax.axis_index("c")
    sid = lax.axis_index("s")
    wid = sid * 2 + cid

    @pl.when(wid == 0)
    def _meta():
        pltpu.sync_copy(rel_hbm, rel_v)
        pltpu.sync_copy(ps_hbm, ps_v)
        pltpu.sync_copy(pm_hbm, pm_v)
        pltpu.sync_copy(meta_hbm, meta_v)
        zero16 = jnp.zeros((16,), jnp.int32)
        neg16 = jnp.full((16,), -1, jnp.int32)
        iota = lax.iota(jnp.int32, 16)
        slot_vec = meta_v[pl.ds(64, 16)]

        for cch in range(4):
            ridx = rel_v[pl.ds(cch * 16, 16)]
            plsc.store_scatter(ps_v, [ridx], zero16)

        tl_vec = meta_v[pl.ds(80, 16)]
        is_slot = iota == slot_vec
        seq_in = meta_v[pl.ds(0, 16)]
        npu_in = meta_v[pl.ds(16, 16)]
        cp_in = meta_v[pl.ds(32, 16)]
        cpp_in = meta_v[pl.ds(48, 16)]
        npn = (tl_vec + (TOKENS_PER_PAGE - 1)) // TOKENS_PER_PAGE
        lpp = jnp.where(tl_vec > 0, (tl_vec - 1) % TOKENS_PER_PAGE, zero16)
        meta_v[pl.ds(0, 16)] = jnp.where(is_slot, tl_vec, seq_in)
        meta_v[pl.ds(16, 16)] = jnp.where(is_slot, npn, npu_in)
        meta_v[pl.ds(32, 16)] = jnp.where(is_slot, neg16, cp_in)
        meta_v[pl.ds(48, 16)] = jnp.where(is_slot, lpp, cpp_in)

        row_v[pl.ds(0, 16)] = neg16
        row_v[pl.ds(16, 16)] = neg16

        def _search(vi, cnt):
            vec = ps_v[pl.ds(pl.multiple_of(vi * 16, 16), 16)]
            zm = vec == 0
            pc = _prefix16(jnp.where(zm, 1, 0).astype(jnp.int32))
            ordn = cnt + pc - 1
            take = zm & (ordn < RESERVE)
            pages = iota + vi * 16
            plsc.store_scatter(row_v, [ordn], pages, mask=take)
            ps_v[pl.ds(pl.multiple_of(vi * 16, 16), 16)] = jnp.where(take, 1, vec)
            hit = zm & (ordn == RESERVE - 1)
            plsc.store_scatter(meta_v, [slot_vec + 32], pages, mask=hit)
            return cnt + _splat_last(pc)

        lax.fori_loop(0, NUM_PAGES // 16, _search, jnp.zeros((16,), jnp.int32))

        base = slot_vec * MAX_PAGES_PER_SLOT + iota
        plsc.store_scatter(pm_v, [base], row_v[pl.ds(0, 16)])
        plsc.store_scatter(pm_v, [base + 16], row_v[pl.ds(16, 16)])
        plsc.store_scatter(pm_v, [base + 32], neg16)
        plsc.store_scatter(pm_v, [base + 48], neg16)

        pltpu.sync_copy(ps_v, ps_out)
        pltpu.sync_copy(pm_v, pm_out)
        pltpu.sync_copy(meta_v.at[pl.ds(0, 64)], meta_out)


@functools.cache
def _get_sc_call():
  mesh = plsc.VectorSubcoreMesh(core_axis_name="c", subcore_axis_name="s")
  return pl.kernel(
    _sc_body,
    out_type=(
        jax.ShapeDtypeStruct((NUM_PAGES,), jnp.int32),
        jax.ShapeDtypeStruct((PM_FLAT,), jnp.int32),
        jax.ShapeDtypeStruct((4 * SLOTS,), jnp.int32),
    ),
    mesh=mesh,
    compiler_params=pltpu.CompilerParams(needs_layout_passes=False),
    scratch_types=[
        pltpu.VMEM((MAX_PAGES_PER_SLOT,), jnp.int32),
        pltpu.VMEM((PM_FLAT,), jnp.int32),
        pltpu.VMEM((NUM_PAGES,), jnp.int32),
        pltpu.VMEM((2 * RESERVE,), jnp.int32),
        pltpu.VMEM((6 * SLOTS,), jnp.int32),
    ],
  )


def kernel(key_pages, value_pages, page_status, page_map, sequence_lengths,
           num_pages_used, current_page, current_page_position, slot, true_length):
    slot = jnp.asarray(slot, jnp.int32)
    tl = jnp.asarray(true_length, jnp.int32)
    released = lax.dynamic_index_in_dim(page_map, slot, axis=0, keepdims=False)
    meta_in = jnp.concatenate([
        sequence_lengths.astype(jnp.int32),
        num_pages_used.astype(jnp.int32),
        current_page.astype(jnp.int32),
        current_page_position.astype(jnp.int32),
        jnp.full((SLOTS,), slot, jnp.int32),
        jnp.full((SLOTS,), tl, jnp.int32),
    ])
    ps_out, pm_out, meta_out = _get_sc_call()(
        page_status, released, page_map.reshape(PM_FLAT), meta_in)

    kt = jnp.transpose(key_pages, (0, 2, 3, 1)).reshape(NROWS, NUM_PAGES)
    vt = jnp.transpose(value_pages, (0, 2, 3, 1)).reshape(NROWS, NUM_PAGES)
    ko, vo = _tc_call(released.reshape(MAX_PAGES_PER_SLOT, 1), kt, vt)
    key_out = jnp.transpose(
        ko.reshape(NUM_KV_HEADS, TOKENS_PER_PAGE, HEAD_DIM, NUM_PAGES),
        (0, 3, 1, 2))
    value_out = jnp.transpose(
        vo.reshape(NUM_KV_HEADS, TOKENS_PER_PAGE, HEAD_DIM, NUM_PAGES),
        (0, 3, 1, 2))

    pm_new = pm_out.reshape(SLOTS, MAX_PAGES_PER_SLOT)
    return (key_out, value_out, ps_out, pm_new,
            meta_out[0:16], meta_out[16:32], meta_out[32:48], meta_out[48:64])


# speedup vs baseline: 1.0733x; 1.0733x over previous
import functools

import jax
import jax.numpy as jnp
from jax import lax
from jax.experimental import pallas as pl
from jax.experimental.pallas import tpu as pltpu
from jax.experimental.pallas import tpu_sc as plsc

NUM_PAGES = 1024
TOKENS_PER_PAGE = 32
SLOTS = 16
MAX_PAGES_PER_SLOT = 64
NUM_KV_HEADS = 8
HEAD_DIM = 64
NROWS = NUM_KV_HEADS * TOKENS_PER_PAGE * HEAD_DIM
PM_FLAT = SLOTS * MAX_PAGES_PER_SLOT
RESERVE = 32
BLK = 1024


def _tc_body(rel_ref, k_ref, v_ref, ko_ref, vo_ref):
    lanes = lax.broadcasted_iota(jnp.int32, (MAX_PAGES_PER_SLOT, NUM_PAGES), 1)
    hit = jnp.max(jnp.where(lanes == rel_ref[...], 1, 0), axis=0, keepdims=True)
    released = hit > 0
    zero = jnp.zeros((), jnp.float32)
    ko_ref[...] = jnp.where(released, zero, k_ref[...])
    vo_ref[...] = jnp.where(released, zero, v_ref[...])


_kv2d = jax.ShapeDtypeStruct((NROWS, NUM_PAGES), jnp.float32)
_tc_call = pl.pallas_call(
    _tc_body,
    out_shape=(_kv2d, _kv2d),
    grid=(NROWS // BLK,),
    in_specs=[
        pl.BlockSpec((MAX_PAGES_PER_SLOT, 1), lambda i: (0, 0)),
        pl.BlockSpec((BLK, NUM_PAGES), lambda i: (i, 0)),
        pl.BlockSpec((BLK, NUM_PAGES), lambda i: (i, 0)),
    ],
    out_specs=[
        pl.BlockSpec((BLK, NUM_PAGES), lambda i: (i, 0)),
        pl.BlockSpec((BLK, NUM_PAGES), lambda i: (i, 0)),
    ],
)


def _gather16(x, idx):
    return x.at[idx].get(mode="promise_in_bounds")


def _prefix16(x):
    lane = lax.iota(jnp.int32, 16)
    zero = jnp.zeros((16,), jnp.int32)
    for k in (1, 2, 4, 8):
        shifted = _gather16(x, jnp.maximum(lane - k, 0))
        x = x + jnp.where(lane >= k, shifted, zero)
    return x


def _splat_last(x):
    return _gather16(x, jnp.full((16,), 15, jnp.int32))


def _sc_body(ps_hbm, rel_hbm, pm_hbm, meta_hbm,
             ps_out, pm_out, meta_out,
             rel_v, pm_v, ps_v, row_v, meta_v):
    cid = lax.axis_index("c")
    sid = lax.axis_index("s")
    wid = sid * 2 + cid

    @pl.when(wid == 0)
    def _meta():
        pltpu.sync_copy(rel_hbm, rel_v)
        pltpu.sync_copy(ps_hbm, ps_v)
        pltpu.sync_copy(pm_hbm, pm_v)
        pltpu.sync_copy(meta_hbm, meta_v)
        zero16 = jnp.zeros((16,), jnp.int32)
        neg16 = jnp.full((16,), -1, jnp.int32)
        iota = lax.iota(jnp.int32, 16)
        slot_vec = meta_v[pl.ds(64, 16)]

        for cch in range(4):
            ridx = rel_v[pl.ds(cch * 16, 16)]
            plsc.store_scatter(ps_v, [ridx], zero16)

        tl_vec = meta_v[pl.ds(80, 16)]
        is_slot = iota == slot_vec
        seq_in = meta_v[pl.ds(0, 16)]
        npu_in = meta_v[pl.ds(16, 16)]
        cp_in = meta_v[pl.ds(32, 16)]
        cpp_in = meta_v[pl.ds(48, 16)]
        npn = (tl_vec + (TOKENS_PER_PAGE - 1)) // TOKENS_PER_PAGE
        lpp = jnp.where(tl_vec > 0, (tl_vec - 1) % TOKENS_PER_PAGE, zero16)
        meta_v[pl.ds(0, 16)] = jnp.where(is_slot, tl_vec, seq_in)
        meta_v[pl.ds(16, 16)] = jnp.where(is_slot, npn, npu_in)
        meta_v[pl.ds(32, 16)] = jnp.where(is_slot, neg16, cp_in)
        meta_v[pl.ds(48, 16)] = jnp.where(is_slot, lpp, cpp_in)

        row_v[pl.ds(0, 16)] = neg16
        row_v[pl.ds(16, 16)] = neg16

        def _search(vi, cnt):
            vec = ps_v[pl.ds(pl.multiple_of(vi * 16, 16), 16)]
            zm = vec == 0
            pc = _prefix16(jnp.where(zm, 1, 0).astype(jnp.int32))
            ordn = cnt + pc - 1
            take = zm & (ordn < RESERVE)
            pages = iota + vi * 16
            plsc.store_scatter(row_v, [ordn], pages, mask=take)
            ps_v[pl.ds(pl.multiple_of(vi * 16, 16), 16)] = jnp.where(take, 1, vec)
            hit = zm & (ordn == RESERVE - 1)
            plsc.store_scatter(meta_v, [slot_vec + 32], pages, mask=hit)
            return cnt + _splat_last(pc)

        lax.fori_loop(0, NUM_PAGES // 16, _search, jnp.zeros((16,), jnp.int32))

        base = slot_vec * MAX_PAGES_PER_SLOT + iota
        plsc.store_scatter(pm_v, [base], row_v[pl.ds(0, 16)])
        plsc.store_scatter(pm_v, [base + 16], row_v[pl.ds(16, 16)])
        plsc.store_scatter(pm_v, [base + 32], neg16)
        plsc.store_scatter(pm_v, [base + 48], neg16)

        pltpu.sync_copy(ps_v, ps_out)
        pltpu.sync_copy(pm_v, pm_out)
        pltpu.sync_copy(meta_v.at[pl.ds(0, 64)], meta_out)


@functools.cache
def _get_sc_call():
  mesh = plsc.VectorSubcoreMesh(core_axis_name="c", subcore_axis_name="s")
  return pl.kernel(
    _sc_body,
    out_type=(
        jax.ShapeDtypeStruct((NUM_PAGES,), jnp.int32),
        jax.ShapeDtypeStruct((PM_FLAT,), jnp.int32),
        jax.ShapeDtypeStruct((4 * SLOTS,), jnp.int32),
    ),
    mesh=mesh,
    compiler_params=pltpu.CompilerParams(needs_layout_passes=False),
    scratch_types=[
        pltpu.VMEM((MAX_PAGES_PER_SLOT,), jnp.int32),
        pltpu.VMEM((PM_FLAT,), jnp.int32),
        pltpu.VMEM((NUM_PAGES,), jnp.int32),
        pltpu.VMEM((2 * RESERVE,), jnp.int32),
        pltpu.VMEM((6 * SLOTS,), jnp.int32),
    ],
  )


def kernel(key_pages, value_pages, page_status, page_map, sequence_lengths,
           num_pages_used, current_page, current_page_position, slot, true_length):
    slot = jnp.asarray(slot, jnp.int32)
    tl = jnp.asarray(true_length, jnp.int32)
    released = lax.dynamic_index_in_dim(page_map, slot, axis=0, keepdims=False)
    meta_in = jnp.concatenate([
        sequence_lengths.astype(jnp.int32),
        num_pages_used.astype(jnp.int32),
        current_page.astype(jnp.int32),
        current_page_position.astype(jnp.int32),
        jnp.full((SLOTS,), slot, jnp.int32),
        jnp.full((SLOTS,), tl, jnp.int32),
    ])
    ps_out, pm_out, meta_out = _get_sc_call()(
        page_status, released, page_map.reshape(PM_FLAT), meta_in)

    kt = jnp.transpose(key_pages, (0, 2, 3, 1)).reshape(NROWS, NUM_PAGES)
    vt = jnp.transpose(value_pages, (0, 2, 3, 1)).reshape(NROWS, NUM_PAGES)
    ko, vo = _tc_call(released.reshape(MAX_PAGES_PER_SLOT, 1), kt, vt)
    key_out = jnp.transpose(
        ko.reshape(NUM_KV_HEADS, TOKENS_PER_PAGE, HEAD_DIM, NUM_PAGES),
        (0, 3, 1, 2))
    value_out = jnp.transpose(
        vo.reshape(NUM_KV_HEADS, TOKENS_PER_PAGE, HEAD_DIM, NUM_PAGES),
        (0, 3, 1, 2))

    pm_new = pm_out.reshape(SLOTS, MAX_PAGES_PER_SLOT)
    return (key_out, value_out, ps_out, pm_new,
            meta_out[0:16], meta_out[16:32], meta_out[32:48], meta_out[48:64])
